# baseline (speedup 1.0000x reference)
import functools

import jax
import jax.numpy as jnp
from jax import lax
from jax.experimental import pallas as pl
from jax.experimental.pallas import tpu as pltpu
from jax.experimental.pallas import tpu_sc as plsc

HIDDEN_DIM = 2048
N_EXPERTS = 64
BLOCK_T = 2048
N_TOKENS = 16384
N_CHUNKS = 4
CHUNK_T = N_TOKENS // N_CHUNKS

NC, NS, L = 2, 16, 16
NW = NC * NS
TW = CHUNK_T // NW
GROUPS = TW // L


def _matmul_block(hs_ref, w_ref, logits_ref):
    logits_ref[...] = lax.dot_general(
        hs_ref[...], w_ref[...], (((1,), (1,)), ((), ())),
        preferred_element_type=jnp.float32)


_sc_mesh = plsc.VectorSubcoreMesh(
    core_axis_name="c", subcore_axis_name="s", num_cores=NC, num_subcores=NS)


@functools.partial(
    pl.kernel,
    mesh=_sc_mesh,
    out_type=(
        jax.ShapeDtypeStruct((CHUNK_T * 2,), jnp.float32),
        jax.ShapeDtypeStruct((CHUNK_T * 2,), jnp.int32),
    ),
    scratch_types=[
        pltpu.VMEM((TW * N_EXPERTS,), jnp.float32),
        pltpu.VMEM((TW * 2,), jnp.float32),
        pltpu.VMEM((TW * 2,), jnp.int32),
    ],
    compiler_params=pltpu.CompilerParams(needs_layout_passes=False),
)
def _sc_top2(logits_hbm, scores_hbm, idx_hbm, slab, sc_v, ix_v):
    wid = lax.axis_index("s") * NC + lax.axis_index("c")
    base = wid * TW
    pltpu.sync_copy(logits_hbm.at[pl.ds(base * N_EXPERTS, TW * N_EXPERTS)], slab)
    iota16 = lax.iota(jnp.int32, L)

    @plsc.parallel_loop(0, GROUPS, unroll=4)
    def _group(g):
        tok = g * L + iota16
        flat = tok * N_EXPERTS
        neg_inf = jnp.full((L,), -jnp.inf, jnp.float32)
        zero_i = jnp.zeros((L,), jnp.int32)
        t1v, t2v = neg_inf, neg_inf
        t1i, t2i = zero_i, zero_i
        for e in range(N_EXPERTS):
            ev = jnp.full((L,), e, jnp.int32)
            v = plsc.load_gather(slab, [flat + e])
            gt1 = v > t1v
            loser = jnp.minimum(v, t1v)
            gt2 = loser > t2v
            cand = jnp.where(gt1, t1i, ev)
            t2v = jnp.maximum(loser, t2v)
            t2i = jnp.where(gt2, cand, t2i)
            t1v = jnp.maximum(v, t1v)
            t1i = jnp.where(gt1, ev, t1i)
        s1 = 1.0 / (1.0 + jnp.exp(t2v - t1v))
        s2 = 1.0 - s1
        two_tok = tok * 2
        plsc.store_scatter(sc_v, [two_tok], s1)
        plsc.store_scatter(sc_v, [two_tok + 1], s2)
        plsc.store_scatter(ix_v, [two_tok], t1i)
        plsc.store_scatter(ix_v, [two_tok + 1], t2i)

    pltpu.sync_copy(sc_v, scores_hbm.at[pl.ds(base * 2, TW * 2)])
    pltpu.sync_copy(ix_v, idx_hbm.at[pl.ds(base * 2, TW * 2)])


def _tc_chunk(hs, weight, chunk):
    nblk = CHUNK_T // BLOCK_T
    return pl.pallas_call(
        _matmul_block,
        grid=(nblk,),
        in_specs=[
            pl.BlockSpec((BLOCK_T, HIDDEN_DIM),
                         lambda i, c=chunk: (c * (CHUNK_T // BLOCK_T) + i, 0)),
            pl.BlockSpec((N_EXPERTS, HIDDEN_DIM), lambda i: (0, 0)),
        ],
        out_specs=pl.BlockSpec((BLOCK_T, N_EXPERTS), lambda i: (i, 0)),
        out_shape=jax.ShapeDtypeStruct((CHUNK_T, N_EXPERTS), jnp.float32),
    )(hs, weight)


@jax.jit
def kernel(hidden_states, weight):
    hs = hidden_states.reshape(-1, HIDDEN_DIM)
    logits_c = [_tc_chunk(hs, weight, c) for c in range(N_CHUNKS)]
    outs = [_sc_top2(lc.reshape(-1)) for lc in logits_c]
    logits = jnp.concatenate(logits_c, axis=0)
    scores = jnp.concatenate([o[0] for o in outs]).reshape(N_TOKENS, 2)
    indices = jnp.concatenate([o[1] for o in outs]).reshape(N_TOKENS, 2)
    return (logits, scores, indices)


# speedup vs baseline: 1.8570x; 1.8570x over previous
import functools

import jax
import jax.numpy as jnp
from jax import lax
from jax.experimental import pallas as pl

HIDDEN_DIM = 2048
N_EXPERTS = 64
BLOCK_T = 2048


def _router_block(hs_ref, w_ref, logits_ref, scores_ref, idx_ref):
    hs = hs_ref[...]
    w = w_ref[...]
    logits = lax.dot_general(hs, w, (((1,), (1,)), ((), ())),
                             preferred_element_type=jnp.float32)
    logits_ref[...] = logits

    e_iota = lax.broadcasted_iota(jnp.int32, logits.shape, 1)
    m1 = jnp.max(logits, axis=-1, keepdims=True)
    i1 = jnp.min(jnp.where(logits == m1, e_iota, N_EXPERTS),
                 axis=-1, keepdims=True)
    masked = jnp.where(e_iota == i1, -jnp.inf, logits)
    m2 = jnp.max(masked, axis=-1, keepdims=True)
    i2 = jnp.min(jnp.where(masked == m2, e_iota, N_EXPERTS),
                 axis=-1, keepdims=True)

    s1 = 1.0 / (1.0 + jnp.exp(m2 - m1))
    s2 = 1.0 - s1
    scores_ref[...] = jnp.concatenate([s1, s2], axis=-1)
    idx_ref[...] = jnp.concatenate([i1, i2], axis=-1)


@jax.jit
def kernel(hidden_states, weight):
    hs = hidden_states.reshape(-1, HIDDEN_DIM)
    n_tokens = hs.shape[0]
    grid = (n_tokens // BLOCK_T,)
    out_shapes = (
        jax.ShapeDtypeStruct((n_tokens, N_EXPERTS), jnp.float32),
        jax.ShapeDtypeStruct((n_tokens, 2), jnp.float32),
        jax.ShapeDtypeStruct((n_tokens, 2), jnp.int32),
    )
    logits, scores, indices = pl.pallas_call(
        _router_block,
        grid=grid,
        in_specs=[
            pl.BlockSpec((BLOCK_T, HIDDEN_DIM), lambda i: (i, 0)),
            pl.BlockSpec((N_EXPERTS, HIDDEN_DIM), lambda i: (0, 0)),
        ],
        out_specs=(
            pl.BlockSpec((BLOCK_T, N_EXPERTS), lambda i: (i, 0)),
            pl.BlockSpec((BLOCK_T, 2), lambda i: (i, 0)),
            pl.BlockSpec((BLOCK_T, 2), lambda i: (i, 0)),
        ),
        out_shape=out_shapes,
    )(hs, weight)
    return (logits, scores, indices)
